# Initial kernel scaffold; baseline (speedup 1.0000x reference)
import jax
import jax.numpy as jnp
from jax.experimental import pallas as pl


def kernel(x, edge_index, W1, b1, W2, b2):
    raise NotImplementedError("write your pallas kernel here")


# speedup vs baseline: 7.7022x; 7.7022x over previous
import functools

import jax
import jax.numpy as jnp
from jax import lax
from jax.experimental import pallas as pl
from jax.experimental.pallas import tpu as pltpu
from jax.experimental.pallas import tpu_sc as plsc

N = 10000
D = 128
NPAD = 10240
NW = 32
NSUB = 16
ROWS_PER_TILE = NPAD // NSUB
CH = 128
CPT = 80
EPT = CH * CPT
EPAD = EPT * NW

_mesh = plsc.VectorSubcoreMesh(core_axis_name="c", subcore_axis_name="s")


@functools.partial(
    pl.kernel,
    out_type=jax.ShapeDtypeStruct((NW, NPAD), jnp.float32),
    mesh=_mesh,
    scratch_types=[
        pltpu.VMEM((NPAD,), jnp.float32),
        pltpu.VMEM((EPT,), jnp.int32),
    ],
    compiler_params=pltpu.CompilerParams(needs_layout_passes=False),
)
def _deg_kernel(dst_hbm, out_hbm, hist_v, didx_v):
    cid = lax.axis_index("c")
    sid = lax.axis_index("s")
    wid = sid * 2 + cid

    zeros16 = jnp.zeros((16,), jnp.float32)

    def zero_body(i, carry):
        hist_v[pl.ds(i * 16, 16)] = zeros16
        return carry

    lax.fori_loop(0, NPAD // 16, zero_body, 0)

    pltpu.sync_copy(dst_hbm.at[pl.ds(wid * EPT, EPT)], didx_v)

    ones16 = jnp.ones((16,), jnp.float32)

    def body(i, carry):
        idx = didx_v[pl.ds(i * 16, 16)]
        plsc.addupdate_scatter(hist_v, [idx], ones16)
        return carry

    lax.fori_loop(0, EPT // 16, body, 0)

    pltpu.sync_copy(hist_v, out_hbm.at[wid])


@functools.partial(
    pl.kernel,
    out_type=jax.ShapeDtypeStruct((2, NPAD, D), jnp.float32),
    mesh=_mesh,
    scratch_types=[
        pltpu.VMEM_SHARED((NPAD, D), jnp.float32),
        pltpu.VMEM((CPT, CH), jnp.int32),
        pltpu.VMEM((CPT, CH), jnp.int32),
        pltpu.VMEM((CH, D), jnp.float32),
        pltpu.SemaphoreType.DMA,
    ],
)
def _agg_kernel(g_hbm, src_hbm, dst_hbm, zrows_hbm, out_hbm,
                acc, sidx, didx, ga, sa):
    cid = lax.axis_index("c")
    sid = lax.axis_index("s")
    wid = sid * 2 + cid

    row0 = sid * ROWS_PER_TILE
    pltpu.sync_copy(zrows_hbm, acc.at[pl.ds(row0, ROWS_PER_TILE)])
    pltpu.sync_copy(src_hbm.at[wid], sidx)
    pltpu.sync_copy(dst_hbm.at[wid], didx)
    plsc.subcore_barrier()

    def body(j, carry):
        pltpu.async_copy(g_hbm.at[sidx.at[j]], ga, sa).wait()
        pltpu.sync_copy(ga, acc.at[didx.at[j]], add=True)
        return carry

    lax.fori_loop(0, CPT, body, 0)

    plsc.subcore_barrier()
    pltpu.sync_copy(acc.at[pl.ds(row0, ROWS_PER_TILE)],
                    out_hbm.at[cid, pl.ds(row0, ROWS_PER_TILE)])


BLK = 1280
GRID = NPAD // BLK


def _row_spec():
    return pl.BlockSpec((BLK, D), lambda i: (i, 0))


def _mm_body(x_ref, w_ref, o_ref):
    o_ref[...] = jnp.dot(x_ref[...], w_ref[...],
                         preferred_element_type=jnp.float32)


_mm = pl.pallas_call(
    _mm_body,
    grid=(GRID,),
    in_specs=[_row_spec(), pl.BlockSpec((D, D), lambda i: (0, 0))],
    out_specs=_row_spec(),
    out_shape=jax.ShapeDtypeStruct((NPAD, D), jnp.float32),
)


def _dinv_body(hist_ref, o_ref):
    deg = jnp.sum(hist_ref[...], axis=0) + 1.0
    o_ref[...] = lax.rsqrt(deg)


_dinv = pl.pallas_call(
    _dinv_body,
    out_shape=jax.ShapeDtypeStruct((NPAD,), jnp.float32),
)


def _scale_body(h_ref, dv_ref, o_ref):
    o_ref[...] = h_ref[...] * dv_ref[...]


_scale = pl.pallas_call(
    _scale_body,
    grid=(GRID,),
    in_specs=[_row_spec(), _row_spec()],
    out_specs=_row_spec(),
    out_shape=jax.ShapeDtypeStruct((NPAD, D), jnp.float32),
)


def _layer2_body(a0_ref, a1_ref, g_ref, dv_ref, b_ref, w_ref, o_ref):
    t = dv_ref[...] * (a0_ref[...] + a1_ref[...] + g_ref[...]) + b_ref[...]
    t = jnp.maximum(t, 0.0)
    o_ref[...] = jnp.dot(t, w_ref[...],
                         preferred_element_type=jnp.float32) * dv_ref[...]


_layer2 = pl.pallas_call(
    _layer2_body,
    grid=(GRID,),
    in_specs=[_row_spec(), _row_spec(), _row_spec(), _row_spec(),
              pl.BlockSpec((1, D), lambda i: (0, 0)),
              pl.BlockSpec((D, D), lambda i: (0, 0))],
    out_specs=_row_spec(),
    out_shape=jax.ShapeDtypeStruct((NPAD, D), jnp.float32),
)


def _fin_body(a0_ref, a1_ref, g_ref, dv_ref, b_ref, o_ref):
    o_ref[...] = (dv_ref[...] * (a0_ref[...] + a1_ref[...] + g_ref[...])
                  + b_ref[...])


_fin = pl.pallas_call(
    _fin_body,
    grid=(GRID,),
    in_specs=[_row_spec(), _row_spec(), _row_spec(), _row_spec(),
              pl.BlockSpec((1, D), lambda i: (0, 0))],
    out_specs=_row_spec(),
    out_shape=jax.ShapeDtypeStruct((NPAD, D), jnp.float32),
)


def kernel(x, edge_index, W1, b1, W2, b2):
    src = edge_index[0].astype(jnp.int32)
    dst = edge_index[1].astype(jnp.int32)
    pad = EPAD - src.shape[0]
    src_p = jnp.concatenate([src, jnp.zeros((pad,), jnp.int32)])
    fill = N + (jnp.arange(pad, dtype=jnp.int32) % (NPAD - N))
    dst_p = jnp.concatenate([dst, fill])
    src3 = src_p.reshape(NW, CPT, CH)
    dst3 = dst_p.reshape(NW, CPT, CH)

    x_p = jnp.pad(x, ((0, NPAD - N), (0, 0)))
    zrows = jnp.zeros((ROWS_PER_TILE, D), jnp.float32)
    b1r = b1.reshape(1, D)
    b2r = b2.reshape(1, D)

    hist = _deg_kernel(dst_p)
    h1 = _mm(x_p, W1)
    dinv = _dinv(hist)
    dinvb = jnp.broadcast_to(dinv[:, None], (NPAD, D))
    g1 = _scale(h1, dinvb)
    acc1 = _agg_kernel(g1, src3, dst3, zrows)
    g2 = _layer2(acc1[0], acc1[1], g1, dinvb, b1r, W2)
    acc2 = _agg_kernel(g2, src3, dst3, zrows)
    out = _fin(acc2[0], acc2[1], g2, dinvb, b2r)
    return out[:N]


# speedup vs baseline: 9.4693x; 1.2294x over previous
import functools

import jax
import jax.numpy as jnp
from jax import lax
from jax.experimental import pallas as pl
from jax.experimental.pallas import tpu as pltpu
from jax.experimental.pallas import tpu_sc as plsc

N = 10000
D = 128
NPAD = 10240
NW = 32
NSUB = 16
ROWS_PER_TILE = NPAD // NSUB
CH = 128
K0 = 118
K1 = 42
KMAX = K0
EPT = CH * (K0 + K1) // 2
EPAD = CH * NSUB * (K0 + K1)

_mesh = plsc.VectorSubcoreMesh(core_axis_name="c", subcore_axis_name="s")


@functools.partial(
    pl.kernel,
    out_type=jax.ShapeDtypeStruct((NW, NPAD), jnp.float32),
    mesh=_mesh,
    scratch_types=[
        pltpu.VMEM((NPAD,), jnp.float32),
        pltpu.VMEM((EPT,), jnp.int32),
    ],
    compiler_params=pltpu.CompilerParams(needs_layout_passes=False),
)
def _deg_kernel(dst_hbm, out_hbm, hist_v, didx_v):
    cid = lax.axis_index("c")
    sid = lax.axis_index("s")
    wid = sid * 2 + cid

    zeros16 = jnp.zeros((16,), jnp.float32)

    def zero_body(i, carry):
        hist_v[pl.ds(i * 16, 16)] = zeros16
        return carry

    lax.fori_loop(0, NPAD // 16, zero_body, 0)

    pltpu.sync_copy(dst_hbm.at[pl.ds(wid * EPT, EPT)], didx_v)

    ones16 = jnp.ones((16,), jnp.float32)

    def body(i, carry):
        idx = didx_v[pl.ds(i * 16, 16)]
        plsc.addupdate_scatter(hist_v, [idx], ones16)
        return carry

    lax.fori_loop(0, EPT // 16, body, 0)

    pltpu.sync_copy(hist_v, out_hbm.at[wid])


@functools.partial(
    pl.kernel,
    out_type=jax.ShapeDtypeStruct((2, NPAD, D), jnp.float32),
    mesh=_mesh,
    scratch_types=[
        pltpu.VMEM_SHARED((NPAD, D), jnp.float32),
        pltpu.VMEM((KMAX, CH), jnp.int32),
        pltpu.VMEM((KMAX, CH), jnp.int32),
        pltpu.VMEM((CH, D), jnp.float32),
        pltpu.SemaphoreType.DMA,
    ],
)
def _agg_kernel(g_hbm, src_hbm, dst_hbm, zrows_hbm, out_hbm,
                acc, sidx, didx, ga, sa):
    cid = lax.axis_index("c")
    sid = lax.axis_index("s")
    wid = sid * 2 + cid

    row0 = sid * ROWS_PER_TILE
    pltpu.sync_copy(zrows_hbm, acc.at[pl.ds(row0, ROWS_PER_TILE)])
    pltpu.sync_copy(src_hbm.at[wid], sidx)
    pltpu.sync_copy(dst_hbm.at[wid], didx)
    plsc.subcore_barrier()

    def body(j, carry):
        pltpu.async_copy(g_hbm.at[sidx.at[j]], ga, sa).wait()
        pltpu.sync_copy(ga, acc.at[didx.at[j]], add=True)
        return carry

    nk = jnp.where(cid == 0, K0, K1)
    lax.fori_loop(0, nk, body, 0)

    plsc.subcore_barrier()
    pltpu.sync_copy(acc.at[pl.ds(row0, ROWS_PER_TILE)],
                    out_hbm.at[cid, pl.ds(row0, ROWS_PER_TILE)])


BLK = 1280
GRID = NPAD // BLK


def _row_spec():
    return pl.BlockSpec((BLK, D), lambda i: (i, 0))


def _mm_body(x_ref, w_ref, o_ref):
    o_ref[...] = jnp.dot(x_ref[...], w_ref[...],
                         preferred_element_type=jnp.float32)


_mm = pl.pallas_call(
    _mm_body,
    grid=(GRID,),
    in_specs=[_row_spec(), pl.BlockSpec((D, D), lambda i: (0, 0))],
    out_specs=_row_spec(),
    out_shape=jax.ShapeDtypeStruct((NPAD, D), jnp.float32),
)


def _dinv_body(hist_ref, o_ref):
    deg = jnp.sum(hist_ref[...], axis=0) + 1.0
    o_ref[...] = lax.rsqrt(deg)


_dinv = pl.pallas_call(
    _dinv_body,
    out_shape=jax.ShapeDtypeStruct((NPAD,), jnp.float32),
)


def _scale_body(h_ref, dv_ref, o_ref):
    o_ref[...] = h_ref[...] * dv_ref[...]


_scale = pl.pallas_call(
    _scale_body,
    grid=(GRID,),
    in_specs=[_row_spec(), _row_spec()],
    out_specs=_row_spec(),
    out_shape=jax.ShapeDtypeStruct((NPAD, D), jnp.float32),
)


def _layer2_body(a0_ref, a1_ref, g_ref, dv_ref, b_ref, w_ref, o_ref):
    t = dv_ref[...] * (a0_ref[...] + a1_ref[...] + g_ref[...]) + b_ref[...]
    t = jnp.maximum(t, 0.0)
    o_ref[...] = jnp.dot(t, w_ref[...],
                         preferred_element_type=jnp.float32) * dv_ref[...]


_layer2 = pl.pallas_call(
    _layer2_body,
    grid=(GRID,),
    in_specs=[_row_spec(), _row_spec(), _row_spec(), _row_spec(),
              pl.BlockSpec((1, D), lambda i: (0, 0)),
              pl.BlockSpec((D, D), lambda i: (0, 0))],
    out_specs=_row_spec(),
    out_shape=jax.ShapeDtypeStruct((NPAD, D), jnp.float32),
)


def _fin_body(a0_ref, a1_ref, g_ref, dv_ref, b_ref, o_ref):
    o_ref[...] = (dv_ref[...] * (a0_ref[...] + a1_ref[...] + g_ref[...])
                  + b_ref[...])


_fin = pl.pallas_call(
    _fin_body,
    grid=(GRID,),
    in_specs=[_row_spec(), _row_spec(), _row_spec(), _row_spec(),
              pl.BlockSpec((1, D), lambda i: (0, 0))],
    out_specs=_row_spec(),
    out_shape=jax.ShapeDtypeStruct((NPAD, D), jnp.float32),
)


def kernel(x, edge_index, W1, b1, W2, b2):
    src = edge_index[0].astype(jnp.int32)
    dst = edge_index[1].astype(jnp.int32)
    pad = EPAD - src.shape[0]
    src_p = jnp.concatenate([src, jnp.zeros((pad,), jnp.int32)])
    fill = N + (jnp.arange(pad, dtype=jnp.int32) % (NPAD - N))
    dst_p = jnp.concatenate([dst, fill])

    def _sched(e):
        c0 = e[:NSUB * K0 * CH].reshape(NSUB, K0, CH)
        c1 = e[NSUB * K0 * CH:].reshape(NSUB, K1, CH)
        c1 = jnp.pad(c1, ((0, 0), (0, KMAX - K1), (0, 0)))
        return jnp.stack([c0, c1], axis=1).reshape(NW, KMAX, CH)

    src3 = _sched(src_p)
    dst3 = _sched(dst_p)

    x_p = jnp.pad(x, ((0, NPAD - N), (0, 0)))
    zrows = jnp.zeros((ROWS_PER_TILE, D), jnp.float32)
    b1r = b1.reshape(1, D)
    b2r = b2.reshape(1, D)

    hist = _deg_kernel(dst_p)
    h1 = _mm(x_p, W1)
    dinv = _dinv(hist)
    dinvb = jnp.broadcast_to(dinv[:, None], (NPAD, D))
    g1 = _scale(h1, dinvb)
    acc1 = _agg_kernel(g1, src3, dst3, zrows)
    g2 = _layer2(acc1[0], acc1[1], g1, dinvb, b1r, W2)
    acc2 = _agg_kernel(g2, src3, dst3, zrows)
    out = _fin(acc2[0], acc2[1], g2, dinvb, b2r)
    return out[:N]


# speedup vs baseline: 9.4790x; 1.0010x over previous
import functools

import jax
import jax.numpy as jnp
from jax import lax
from jax.experimental import pallas as pl
from jax.experimental.pallas import tpu as pltpu
from jax.experimental.pallas import tpu_sc as plsc

N = 10000
D = 128
NPAD = 10240
NW = 32
NSUB = 16
ROWS_PER_TILE = NPAD // NSUB
CH = 128
K0 = 118
K1 = 42
KMAX = K0
EPT = CH * (K0 + K1) // 2
EPAD = CH * NSUB * (K0 + K1)

_mesh = plsc.VectorSubcoreMesh(core_axis_name="c", subcore_axis_name="s")


@functools.partial(
    pl.kernel,
    out_type=jax.ShapeDtypeStruct((NW, NPAD), jnp.float32),
    mesh=_mesh,
    scratch_types=[
        pltpu.VMEM((NPAD,), jnp.float32),
        pltpu.VMEM((EPT,), jnp.int32),
    ],
    compiler_params=pltpu.CompilerParams(needs_layout_passes=False),
)
def _deg_kernel(dst_hbm, out_hbm, hist_v, didx_v):
    cid = lax.axis_index("c")
    sid = lax.axis_index("s")
    wid = sid * 2 + cid

    zeros16 = jnp.zeros((16,), jnp.float32)

    def zero_body(i, carry):
        hist_v[pl.ds(i * 16, 16)] = zeros16
        return carry

    lax.fori_loop(0, NPAD // 16, zero_body, 0)

    pltpu.sync_copy(dst_hbm.at[pl.ds(wid * EPT, EPT)], didx_v)

    ones16 = jnp.ones((16,), jnp.float32)

    def body(i, carry):
        idx = didx_v[pl.ds(i * 16, 16)]
        plsc.addupdate_scatter(hist_v, [idx], ones16)
        return carry

    lax.fori_loop(0, EPT // 16, body, 0)

    pltpu.sync_copy(hist_v, out_hbm.at[wid])


@functools.partial(
    pl.kernel,
    out_type=jax.ShapeDtypeStruct((2, NPAD, D), jnp.float32),
    mesh=_mesh,
    scratch_types=[
        pltpu.VMEM_SHARED((NPAD, D), jnp.float32),
        pltpu.VMEM((2, CH), jnp.int32),
        pltpu.VMEM((2, CH), jnp.int32),
        pltpu.VMEM((CH, D), jnp.float32),
        pltpu.VMEM((CH, D), jnp.float32),
        pltpu.SemaphoreType.DMA,
        pltpu.SemaphoreType.DMA,
    ],
)
def _agg_kernel(g_hbm, e_hbm, zrows_hbm, out_hbm,
                acc, ia, ib, ga, gb, sa, sb):
    cid = lax.axis_index("c")
    sid = lax.axis_index("s")
    wid = sid * 2 + cid

    row0 = sid * ROWS_PER_TILE
    pltpu.sync_copy(zrows_hbm, acc.at[pl.ds(row0, ROWS_PER_TILE)])
    plsc.subcore_barrier()

    nk = jnp.where(cid == 0, K0, K1)
    pltpu.sync_copy(e_hbm.at[wid, 0], ia)
    pltpu.async_copy(g_hbm.at[ia.at[0]], ga, sa)
    pltpu.sync_copy(e_hbm.at[wid, 1], ib)
    pltpu.async_copy(g_hbm.at[ib.at[0]], gb, sb)

    def body(k, carry):
        j0 = 2 * k
        j1 = 2 * k + 1
        pltpu.make_async_copy(g_hbm.at[ia.at[0]], ga, sa).wait()
        pltpu.sync_copy(ga, acc.at[ia.at[1]], add=True)

        @pl.when(j0 + 2 < nk)
        def _():
            pltpu.sync_copy(e_hbm.at[wid, j0 + 2], ia)
            pltpu.async_copy(g_hbm.at[ia.at[0]], ga, sa)

        pltpu.make_async_copy(g_hbm.at[ib.at[0]], gb, sb).wait()
        pltpu.sync_copy(gb, acc.at[ib.at[1]], add=True)

        @pl.when(j1 + 2 < nk)
        def _():
            pltpu.sync_copy(e_hbm.at[wid, j1 + 2], ib)
            pltpu.async_copy(g_hbm.at[ib.at[0]], gb, sb)

        return carry

    lax.fori_loop(0, nk // 2, body, 0)

    plsc.subcore_barrier()
    pltpu.sync_copy(acc.at[pl.ds(row0, ROWS_PER_TILE)],
                    out_hbm.at[cid, pl.ds(row0, ROWS_PER_TILE)])


BLK = 1280
GRID = NPAD // BLK


def _row_spec():
    return pl.BlockSpec((BLK, D), lambda i: (i, 0))


def _mm_body(x_ref, w_ref, o_ref):
    o_ref[...] = jnp.dot(x_ref[...], w_ref[...],
                         preferred_element_type=jnp.float32)


_mm = pl.pallas_call(
    _mm_body,
    grid=(GRID,),
    in_specs=[_row_spec(), pl.BlockSpec((D, D), lambda i: (0, 0))],
    out_specs=_row_spec(),
    out_shape=jax.ShapeDtypeStruct((NPAD, D), jnp.float32),
)


def _dinv_body(hist_ref, o_ref):
    deg = jnp.sum(hist_ref[...], axis=0) + 1.0
    o_ref[...] = lax.rsqrt(deg)


_dinv = pl.pallas_call(
    _dinv_body,
    out_shape=jax.ShapeDtypeStruct((NPAD,), jnp.float32),
)


def _scale_body(h_ref, dv_ref, o_ref):
    o_ref[...] = h_ref[...] * dv_ref[...]


_scale = pl.pallas_call(
    _scale_body,
    grid=(GRID,),
    in_specs=[_row_spec(), _row_spec()],
    out_specs=_row_spec(),
    out_shape=jax.ShapeDtypeStruct((NPAD, D), jnp.float32),
)


def _layer2_body(a0_ref, a1_ref, g_ref, dv_ref, b_ref, w_ref, o_ref):
    t = dv_ref[...] * (a0_ref[...] + a1_ref[...] + g_ref[...]) + b_ref[...]
    t = jnp.maximum(t, 0.0)
    o_ref[...] = jnp.dot(t, w_ref[...],
                         preferred_element_type=jnp.float32) * dv_ref[...]


_layer2 = pl.pallas_call(
    _layer2_body,
    grid=(GRID,),
    in_specs=[_row_spec(), _row_spec(), _row_spec(), _row_spec(),
              pl.BlockSpec((1, D), lambda i: (0, 0)),
              pl.BlockSpec((D, D), lambda i: (0, 0))],
    out_specs=_row_spec(),
    out_shape=jax.ShapeDtypeStruct((NPAD, D), jnp.float32),
)


def _fin_body(a0_ref, a1_ref, g_ref, dv_ref, b_ref, o_ref):
    o_ref[...] = (dv_ref[...] * (a0_ref[...] + a1_ref[...] + g_ref[...])
                  + b_ref[...])


_fin = pl.pallas_call(
    _fin_body,
    grid=(GRID,),
    in_specs=[_row_spec(), _row_spec(), _row_spec(), _row_spec(),
              pl.BlockSpec((1, D), lambda i: (0, 0))],
    out_specs=_row_spec(),
    out_shape=jax.ShapeDtypeStruct((NPAD, D), jnp.float32),
)


def kernel(x, edge_index, W1, b1, W2, b2):
    src = edge_index[0].astype(jnp.int32)
    dst = edge_index[1].astype(jnp.int32)
    pad = EPAD - src.shape[0]
    src_p = jnp.concatenate([src, jnp.zeros((pad,), jnp.int32)])
    fill = N + (jnp.arange(pad, dtype=jnp.int32) % (NPAD - N))
    dst_p = jnp.concatenate([dst, fill])

    def _sched(e):
        c0 = e[:NSUB * K0 * CH].reshape(NSUB, K0, CH)
        c1 = e[NSUB * K0 * CH:].reshape(NSUB, K1, CH)
        c1 = jnp.pad(c1, ((0, 0), (0, KMAX - K1), (0, 0)))
        return jnp.stack([c0, c1], axis=1).reshape(NW, KMAX, CH)

    e4 = jnp.stack([_sched(src_p), _sched(dst_p)], axis=2)

    x_p = jnp.pad(x, ((0, NPAD - N), (0, 0)))
    zrows = jnp.zeros((ROWS_PER_TILE, D), jnp.float32)
    b1r = b1.reshape(1, D)
    b2r = b2.reshape(1, D)

    hist = _deg_kernel(dst_p)
    h1 = _mm(x_p, W1)
    dinv = _dinv(hist)
    dinvb = jnp.broadcast_to(dinv[:, None], (NPAD, D))
    g1 = _scale(h1, dinvb)
    acc1 = _agg_kernel(g1, e4, zrows)
    g2 = _layer2(acc1[0], acc1[1], g1, dinvb, b1r, W2)
    acc2 = _agg_kernel(g2, e4, zrows)
    out = _fin(acc2[0], acc2[1], g2, dinvb, b2r)
    return out[:N]


# speedup vs baseline: 27.2586x; 2.8757x over previous
import functools

import numpy as np

import jax
import jax.numpy as jnp
from jax import lax
from jax.experimental import pallas as pl
from jax.experimental.pallas import tpu as pltpu
from jax.experimental.pallas import tpu_sc as plsc

N = 10000
D = 128
NPAD = 10240
NW = 32
NSUB = 16
ROWS_PER_TILE = NPAD // NSUB
CH = 128
K = 80
EPT = CH * K
EPAD = EPT * NW
NCH = EPAD // CH
NCH_REAL = 320000 // CH

def _build_perm():
    reals = list(range(NCH_REAL))
    counts = [79 if t < 4 else 78 for t in range(NW)]
    perm = []
    ri, pi = 0, NCH_REAL
    for t in range(NW):
        c = counts[t]
        perm += reals[ri:ri + c]
        ri += c
        perm += list(range(pi, pi + K - c))
        pi += K - c
    return np.asarray(perm, dtype=np.int32)


_PERM = _build_perm()

_mesh = plsc.VectorSubcoreMesh(core_axis_name="c", subcore_axis_name="s")


@functools.partial(
    pl.kernel,
    out_type=jax.ShapeDtypeStruct((NW, NPAD), jnp.float32),
    mesh=_mesh,
    scratch_types=[
        pltpu.VMEM((NPAD,), jnp.float32),
        pltpu.VMEM((EPT,), jnp.int32),
    ],
    compiler_params=pltpu.CompilerParams(needs_layout_passes=False),
)
def _deg_kernel(dst_hbm, out_hbm, hist_v, didx_v):
    cid = lax.axis_index("c")
    sid = lax.axis_index("s")
    wid = sid * 2 + cid

    zeros16 = jnp.zeros((16,), jnp.float32)

    def zero_body(i, carry):
        hist_v[pl.ds(i * 16, 16)] = zeros16
        return carry

    lax.fori_loop(0, NPAD // 16, zero_body, 0)

    pltpu.sync_copy(dst_hbm.at[pl.ds(wid * EPT, EPT)], didx_v)

    ones16 = jnp.ones((16,), jnp.float32)

    def body(i, carry):
        idx = didx_v[pl.ds(i * 16, 16)]
        plsc.addupdate_scatter(hist_v, [idx], ones16)
        return carry

    lax.fori_loop(0, EPT // 16, body, 0)

    pltpu.sync_copy(hist_v, out_hbm.at[wid])


@functools.partial(
    pl.kernel,
    out_type=jax.ShapeDtypeStruct((2, NPAD, D), jnp.float32),
    mesh=_mesh,
    scratch_types=[
        pltpu.VMEM_SHARED((NPAD, D), jnp.float32),
        pltpu.VMEM((2, CH), jnp.int32),
        pltpu.VMEM((2, CH), jnp.int32),
        pltpu.VMEM((CH, D), jnp.float32),
        pltpu.VMEM((CH, D), jnp.float32),
        pltpu.SemaphoreType.DMA,
        pltpu.SemaphoreType.DMA,
    ],
)
def _agg_kernel(g_hbm, e_hbm, zrows_hbm, out_hbm,
                acc, ia, ib, ga, gb, sa, sb):
    cid = lax.axis_index("c")
    sid = lax.axis_index("s")
    wid = sid * 2 + cid

    row0 = sid * ROWS_PER_TILE
    pltpu.sync_copy(zrows_hbm, acc.at[pl.ds(row0, ROWS_PER_TILE)])
    plsc.subcore_barrier()

    pltpu.sync_copy(e_hbm.at[wid, 0], ia)
    pltpu.async_copy(g_hbm.at[ia.at[0]], ga, sa)
    pltpu.sync_copy(e_hbm.at[wid, 1], ib)
    pltpu.async_copy(g_hbm.at[ib.at[0]], gb, sb)

    def body(k, carry):
        j0 = 2 * k
        j1 = 2 * k + 1
        pltpu.make_async_copy(g_hbm.at[ia.at[0]], ga, sa).wait()
        pltpu.sync_copy(ga, acc.at[ia.at[1]], add=True)

        @pl.when(j0 + 2 < K)
        def _():
            pltpu.sync_copy(e_hbm.at[wid, j0 + 2], ia)
            pltpu.async_copy(g_hbm.at[ia.at[0]], ga, sa)

        pltpu.make_async_copy(g_hbm.at[ib.at[0]], gb, sb).wait()
        pltpu.sync_copy(gb, acc.at[ib.at[1]], add=True)

        @pl.when(j1 + 2 < K)
        def _():
            pltpu.sync_copy(e_hbm.at[wid, j1 + 2], ib)
            pltpu.async_copy(g_hbm.at[ib.at[0]], gb, sb)

        return carry

    lax.fori_loop(0, K // 2, body, 0)

    plsc.subcore_barrier()
    pltpu.sync_copy(acc.at[pl.ds(row0, ROWS_PER_TILE)],
                    out_hbm.at[cid, pl.ds(row0, ROWS_PER_TILE)])


BLK = 1280
GRID = NPAD // BLK


def _row_spec():
    return pl.BlockSpec((BLK, D), lambda i: (i, 0))


def _mm_body(x_ref, w_ref, o_ref):
    o_ref[...] = jnp.dot(x_ref[...], w_ref[...],
                         preferred_element_type=jnp.float32)


_mm = pl.pallas_call(
    _mm_body,
    grid=(GRID,),
    in_specs=[_row_spec(), pl.BlockSpec((D, D), lambda i: (0, 0))],
    out_specs=_row_spec(),
    out_shape=jax.ShapeDtypeStruct((NPAD, D), jnp.float32),
)


def _dinv_body(hist_ref, o_ref):
    deg = jnp.sum(hist_ref[...], axis=0) + 1.0
    o_ref[...] = lax.rsqrt(deg)


_dinv = pl.pallas_call(
    _dinv_body,
    out_shape=jax.ShapeDtypeStruct((NPAD,), jnp.float32),
)


def _scale_body(h_ref, dv_ref, o_ref):
    o_ref[...] = h_ref[...] * dv_ref[...]


_scale = pl.pallas_call(
    _scale_body,
    grid=(GRID,),
    in_specs=[_row_spec(), _row_spec()],
    out_specs=_row_spec(),
    out_shape=jax.ShapeDtypeStruct((NPAD, D), jnp.float32),
)


def _layer2_body(a0_ref, a1_ref, g_ref, dv_ref, b_ref, w_ref, o_ref):
    t = dv_ref[...] * (a0_ref[...] + a1_ref[...] + g_ref[...]) + b_ref[...]
    t = jnp.maximum(t, 0.0)
    o_ref[...] = jnp.dot(t, w_ref[...],
                         preferred_element_type=jnp.float32) * dv_ref[...]


_layer2 = pl.pallas_call(
    _layer2_body,
    grid=(GRID,),
    in_specs=[_row_spec(), _row_spec(), _row_spec(), _row_spec(),
              pl.BlockSpec((1, D), lambda i: (0, 0)),
              pl.BlockSpec((D, D), lambda i: (0, 0))],
    out_specs=_row_spec(),
    out_shape=jax.ShapeDtypeStruct((NPAD, D), jnp.float32),
)


def _fin_body(a0_ref, a1_ref, g_ref, dv_ref, b_ref, o_ref):
    o_ref[...] = (dv_ref[...] * (a0_ref[...] + a1_ref[...] + g_ref[...])
                  + b_ref[...])


_fin = pl.pallas_call(
    _fin_body,
    grid=(GRID,),
    in_specs=[_row_spec(), _row_spec(), _row_spec(), _row_spec(),
              pl.BlockSpec((1, D), lambda i: (0, 0))],
    out_specs=_row_spec(),
    out_shape=jax.ShapeDtypeStruct((NPAD, D), jnp.float32),
)


def kernel(x, edge_index, W1, b1, W2, b2):
    src = edge_index[0].astype(jnp.int32)
    dst = edge_index[1].astype(jnp.int32)
    pad = EPAD - src.shape[0]
    src_p = jnp.concatenate([src, jnp.arange(pad, dtype=jnp.int32) % N])
    fill = N + (jnp.arange(pad, dtype=jnp.int32) % (NPAD - N))
    dst_p = jnp.concatenate([dst, fill])

    def _sched(e):
        return e.reshape(NCH, CH)[_PERM].reshape(NW, K, CH)

    e4 = jnp.stack([_sched(src_p), _sched(dst_p)], axis=2)

    x_p = jnp.pad(x, ((0, NPAD - N), (0, 0)))
    zrows = jnp.zeros((ROWS_PER_TILE, D), jnp.float32)
    b1r = b1.reshape(1, D)
    b2r = b2.reshape(1, D)

    hist = _deg_kernel(dst_p)
    h1 = _mm(x_p, W1)
    dinv = _dinv(hist)
    dinvb = jnp.broadcast_to(dinv[:, None], (NPAD, D))
    g1 = _scale(h1, dinvb)
    acc1 = _agg_kernel(g1, e4, zrows)
    g2 = _layer2(acc1[0], acc1[1], g1, dinvb, b1r, W2)
    acc2 = _agg_kernel(g2, e4, zrows)
    out = _fin(acc2[0], acc2[1], g2, dinvb, b2r)
    return out[:N]


# speedup vs baseline: 28.6869x; 1.0524x over previous
import functools

import numpy as np

import jax
import jax.numpy as jnp
from jax import lax
from jax.experimental import pallas as pl
from jax.experimental.pallas import tpu as pltpu
from jax.experimental.pallas import tpu_sc as plsc

N = 10000
D = 128
NPAD = 10240
NW = 32
NSUB = 16
ROWS_PER_TILE = NPAD // NSUB
CH = 128
K = 80
EPT = CH * K
EPAD = EPT * NW
NCH = EPAD // CH
NCH_REAL = 320000 // CH

def _build_perm():
    reals = list(range(NCH_REAL))
    counts = [79 if t < 4 else 78 for t in range(NW)]
    perm = []
    ri, pi = 0, NCH_REAL
    for t in range(NW):
        c = counts[t]
        perm += reals[ri:ri + c]
        ri += c
        perm += list(range(pi, pi + K - c))
        pi += K - c
    return np.asarray(perm, dtype=np.int32)


_PERM = _build_perm()

_mesh = plsc.VectorSubcoreMesh(core_axis_name="c", subcore_axis_name="s")


@functools.partial(
    pl.kernel,
    out_type=jax.ShapeDtypeStruct((NW, NPAD), jnp.float32),
    mesh=_mesh,
    scratch_types=[
        pltpu.VMEM((NPAD,), jnp.float32),
        pltpu.VMEM((EPT,), jnp.int32),
    ],
    compiler_params=pltpu.CompilerParams(needs_layout_passes=False),
)
def _deg_kernel(dst_hbm, out_hbm, hist_v, didx_v):
    cid = lax.axis_index("c")
    sid = lax.axis_index("s")
    wid = sid * 2 + cid

    zeros16 = jnp.zeros((16,), jnp.float32)

    def zero_body(i, carry):
        hist_v[pl.ds(i * 16, 16)] = zeros16
        return carry

    lax.fori_loop(0, NPAD // 16, zero_body, 0)

    pltpu.sync_copy(dst_hbm.at[pl.ds(wid * EPT, EPT)], didx_v)

    ones16 = jnp.ones((16,), jnp.float32)

    def body(i, carry):
        idx = didx_v[pl.ds(i * 16, 16)]
        plsc.addupdate_scatter(hist_v, [idx], ones16)
        return carry

    lax.fori_loop(0, EPT // 16, body, 0)

    pltpu.sync_copy(hist_v, out_hbm.at[wid])


@functools.partial(
    pl.kernel,
    out_type=jax.ShapeDtypeStruct((2, NPAD, D), jnp.float32),
    mesh=_mesh,
    scratch_types=[
        pltpu.VMEM_SHARED((NPAD, D), jnp.float32),
        pltpu.VMEM((2, CH), jnp.int32),
        pltpu.VMEM((2, CH), jnp.int32),
        pltpu.VMEM((CH, D), jnp.float32),
        pltpu.VMEM((CH, D), jnp.float32),
        pltpu.SemaphoreType.DMA,
        pltpu.SemaphoreType.DMA,
    ],
)
def _agg_kernel(g_hbm, e_hbm, zrows_hbm, out_hbm,
                acc, ia, ib, ga, gb, sa, sb):
    cid = lax.axis_index("c")
    sid = lax.axis_index("s")
    wid = sid * 2 + cid

    row0 = sid * ROWS_PER_TILE
    pltpu.sync_copy(zrows_hbm, acc.at[pl.ds(row0, ROWS_PER_TILE)])
    plsc.subcore_barrier()

    pltpu.sync_copy(e_hbm.at[wid, 0], ia)
    pltpu.async_copy(g_hbm.at[ia.at[0]], ga, sa)
    pltpu.sync_copy(e_hbm.at[wid, 1], ib)
    pltpu.async_copy(g_hbm.at[ib.at[0]], gb, sb)

    def body(k, carry):
        j0 = 2 * k
        j1 = 2 * k + 1
        pltpu.make_async_copy(g_hbm.at[ia.at[0]], ga, sa).wait()
        pltpu.sync_copy(ga, acc.at[ia.at[1]], add=True)

        @pl.when(j0 + 2 < K)
        def _():
            pltpu.sync_copy(e_hbm.at[wid, j0 + 2], ia)
            pltpu.async_copy(g_hbm.at[ia.at[0]], ga, sa)

        pltpu.make_async_copy(g_hbm.at[ib.at[0]], gb, sb).wait()
        pltpu.sync_copy(gb, acc.at[ib.at[1]], add=True)

        @pl.when(j1 + 2 < K)
        def _():
            pltpu.sync_copy(e_hbm.at[wid, j1 + 2], ib)
            pltpu.async_copy(g_hbm.at[ib.at[0]], gb, sb)

        return carry

    lax.fori_loop(0, K // 2, body, 0)

    plsc.subcore_barrier()
    pltpu.sync_copy(acc.at[pl.ds(row0, ROWS_PER_TILE)],
                    out_hbm.at[cid, pl.ds(row0, ROWS_PER_TILE)])


BLKR = 1000
GRIDR = N // BLKR


def _node_spec():
    return pl.BlockSpec((BLKR, D), lambda i: (i, 0))


def _ht_spec():
    return pl.BlockSpec((BLKR, NW), lambda i: (i, 0))


def _acc_spec(part):
    return pl.BlockSpec((1, BLKR, D), lambda i, _p=part: (_p, i, 0))


def _dinv_col(ht_ref):
    return lax.rsqrt(jnp.sum(ht_ref[...], axis=1, keepdims=True) + 1.0)


def _mm1_body(ht_ref, x_ref, w_ref, o_ref):
    o_ref[...] = jnp.dot(x_ref[...], w_ref[...],
                         preferred_element_type=jnp.float32) * _dinv_col(ht_ref)


_mm1 = pl.pallas_call(
    _mm1_body,
    grid=(GRIDR,),
    in_specs=[_ht_spec(), _node_spec(), pl.BlockSpec((D, D), lambda i: (0, 0))],
    out_specs=_node_spec(),
    out_shape=jax.ShapeDtypeStruct((N, D), jnp.float32),
)


def _layer2_body(ht_ref, a0_ref, a1_ref, g_ref, b_ref, w_ref, o_ref):
    dv = _dinv_col(ht_ref)
    t = dv * (a0_ref[0] + a1_ref[0] + g_ref[...]) + b_ref[...]
    t = jnp.maximum(t, 0.0)
    o_ref[...] = jnp.dot(t, w_ref[...],
                         preferred_element_type=jnp.float32) * dv


_layer2 = pl.pallas_call(
    _layer2_body,
    grid=(GRIDR,),
    in_specs=[_ht_spec(), _acc_spec(0), _acc_spec(1), _node_spec(),
              pl.BlockSpec((1, D), lambda i: (0, 0)),
              pl.BlockSpec((D, D), lambda i: (0, 0))],
    out_specs=_node_spec(),
    out_shape=jax.ShapeDtypeStruct((N, D), jnp.float32),
)


def _fin_body(ht_ref, a0_ref, a1_ref, g_ref, b_ref, o_ref):
    dv = _dinv_col(ht_ref)
    o_ref[...] = dv * (a0_ref[0] + a1_ref[0] + g_ref[...]) + b_ref[...]


_fin = pl.pallas_call(
    _fin_body,
    grid=(GRIDR,),
    in_specs=[_ht_spec(), _acc_spec(0), _acc_spec(1), _node_spec(),
              pl.BlockSpec((1, D), lambda i: (0, 0))],
    out_specs=_node_spec(),
    out_shape=jax.ShapeDtypeStruct((N, D), jnp.float32),
)


def kernel(x, edge_index, W1, b1, W2, b2):
    src = edge_index[0].astype(jnp.int32)
    dst = edge_index[1].astype(jnp.int32)
    pad = EPAD - src.shape[0]
    src_p = jnp.concatenate([src, jnp.arange(pad, dtype=jnp.int32) % N])
    fill = N + (jnp.arange(pad, dtype=jnp.int32) % (NPAD - N))
    dst_p = jnp.concatenate([dst, fill])

    def _sched(e):
        return e.reshape(NCH, CH)[_PERM].reshape(NW, K, CH)

    e4 = jnp.stack([_sched(src_p), _sched(dst_p)], axis=2)

    zrows = jnp.zeros((ROWS_PER_TILE, D), jnp.float32)
    b1r = b1.reshape(1, D)
    b2r = b2.reshape(1, D)

    hist = _deg_kernel(dst_p)
    ht = hist.T
    g1 = _mm1(ht, x, W1)
    acc1 = _agg_kernel(g1, e4, zrows)
    g2 = _layer2(ht, acc1, acc1, g1, b1r, W2)
    acc2 = _agg_kernel(g2, e4, zrows)
    return _fin(ht, acc2, acc2, g2, b2r)


# speedup vs baseline: 31.4658x; 1.0969x over previous
import functools

import numpy as np

import jax
import jax.numpy as jnp
from jax import lax
from jax.experimental import pallas as pl
from jax.experimental.pallas import tpu as pltpu
from jax.experimental.pallas import tpu_sc as plsc

N = 10000
D = 128
NPAD = 10240
NW = 32
NSUB = 16
ROWS_PER_TILE = NPAD // NSUB
CH = 128
K = 80
BCH = 16
NB = K // BCH
EPT = CH * K
EPAD = EPT * NW
NCH = EPAD // CH
NCH_REAL = 320000 // CH

def _build_perm():
    reals = list(range(NCH_REAL))
    counts = [79 if t < 4 else 78 for t in range(NW)]
    perm = []
    ri, pi = 0, NCH_REAL
    for t in range(NW):
        c = counts[t]
        perm += reals[ri:ri + c]
        ri += c
        perm += list(range(pi, pi + K - c))
        pi += K - c
    return np.asarray(perm, dtype=np.int32)


_PERM = _build_perm()

_mesh = plsc.VectorSubcoreMesh(core_axis_name="c", subcore_axis_name="s")


@functools.partial(
    pl.kernel,
    out_type=jax.ShapeDtypeStruct((NW, NPAD), jnp.float32),
    mesh=_mesh,
    scratch_types=[
        pltpu.VMEM((NPAD,), jnp.float32),
        pltpu.VMEM((EPT,), jnp.int32),
    ],
    compiler_params=pltpu.CompilerParams(needs_layout_passes=False),
)
def _deg_kernel(dst_hbm, out_hbm, hist_v, didx_v):
    cid = lax.axis_index("c")
    sid = lax.axis_index("s")
    wid = sid * 2 + cid

    zeros16 = jnp.zeros((16,), jnp.float32)

    def zero_body(i, carry):
        hist_v[pl.ds(i * 16, 16)] = zeros16
        return carry

    lax.fori_loop(0, NPAD // 16, zero_body, 0)

    pltpu.sync_copy(dst_hbm.at[pl.ds(wid * EPT, EPT)], didx_v)

    ones16 = jnp.ones((16,), jnp.float32)

    def body(i, carry):
        idx = didx_v[pl.ds(i * 16, 16)]
        plsc.addupdate_scatter(hist_v, [idx], ones16)
        return carry

    lax.fori_loop(0, EPT // 16, body, 0)

    pltpu.sync_copy(hist_v, out_hbm.at[wid])


@functools.partial(
    pl.kernel,
    out_type=jax.ShapeDtypeStruct((2, NPAD, D), jnp.float32),
    mesh=_mesh,
    scratch_types=[
        pltpu.VMEM_SHARED((NPAD, D), jnp.float32),
        pltpu.VMEM((2, BCH, 2, CH), jnp.int32),
        pltpu.VMEM((CH, D), jnp.float32),
        pltpu.VMEM((CH, D), jnp.float32),
        pltpu.SemaphoreType.DMA,
        pltpu.SemaphoreType.DMA,
        pltpu.SemaphoreType.DMA,
    ],
)
def _agg_kernel(g_hbm, e_hbm, zrows_hbm, out_hbm,
                acc, eb, ga, gb, se, sa, sb):
    cid = lax.axis_index("c")
    sid = lax.axis_index("s")
    wid = sid * 2 + cid

    row0 = sid * ROWS_PER_TILE
    pltpu.sync_copy(zrows_hbm, acc.at[pl.ds(row0, ROWS_PER_TILE)])
    plsc.subcore_barrier()

    pltpu.sync_copy(e_hbm.at[wid, 0], eb.at[0])
    pltpu.async_copy(e_hbm.at[wid, 1], eb.at[1], se)
    pltpu.async_copy(g_hbm.at[eb.at[0, 0, 0]], ga, sa)
    pltpu.async_copy(g_hbm.at[eb.at[0, 1, 0]], gb, sb)

    def outer(b, carry):
        slot = lax.rem(b, 2)
        nslot = 1 - slot
        for p in range(BCH // 2):
            j0 = 2 * p
            j1 = 2 * p + 1
            pltpu.make_async_copy(g_hbm.at[eb.at[slot, j0, 0]], ga, sa).wait()
            pltpu.sync_copy(ga, acc.at[eb.at[slot, j0, 1]], add=True)
            if j0 + 2 < BCH:
                pltpu.async_copy(g_hbm.at[eb.at[slot, j0 + 2, 0]], ga, sa)
            else:
                @pl.when(b + 1 < NB)
                def _():
                    pltpu.make_async_copy(e_hbm.at[wid, b + 1],
                                          eb.at[nslot], se).wait()
                    pltpu.async_copy(g_hbm.at[eb.at[nslot, 0, 0]], ga, sa)

            pltpu.make_async_copy(g_hbm.at[eb.at[slot, j1, 0]], gb, sb).wait()
            pltpu.sync_copy(gb, acc.at[eb.at[slot, j1, 1]], add=True)
            if j1 + 2 < BCH:
                pltpu.async_copy(g_hbm.at[eb.at[slot, j1 + 2, 0]], gb, sb)
            else:
                @pl.when(b + 1 < NB)
                def _():
                    pltpu.async_copy(g_hbm.at[eb.at[nslot, 1, 0]], gb, sb)

                @pl.when(b + 2 < NB)
                def _():
                    pltpu.async_copy(e_hbm.at[wid, b + 2], eb.at[slot], se)

        return carry

    lax.fori_loop(0, NB, outer, 0)

    plsc.subcore_barrier()
    pltpu.sync_copy(acc.at[pl.ds(row0, ROWS_PER_TILE)],
                    out_hbm.at[cid, pl.ds(row0, ROWS_PER_TILE)])


BLKR = 1000
GRIDR = N // BLKR


def _node_spec():
    return pl.BlockSpec((BLKR, D), lambda i: (i, 0))


def _ht_spec():
    return pl.BlockSpec((BLKR, NW), lambda i: (i, 0))


def _acc_spec(part):
    return pl.BlockSpec((1, BLKR, D), lambda i, _p=part: (_p, i, 0))


def _dinv_col(ht_ref):
    return lax.rsqrt(jnp.sum(ht_ref[...], axis=1, keepdims=True) + 1.0)


def _mm1_body(ht_ref, x_ref, w_ref, o_ref):
    o_ref[...] = jnp.dot(x_ref[...], w_ref[...],
                         preferred_element_type=jnp.float32) * _dinv_col(ht_ref)


_mm1 = pl.pallas_call(
    _mm1_body,
    grid=(GRIDR,),
    in_specs=[_ht_spec(), _node_spec(), pl.BlockSpec((D, D), lambda i: (0, 0))],
    out_specs=_node_spec(),
    out_shape=jax.ShapeDtypeStruct((N, D), jnp.float32),
)


def _layer2_body(ht_ref, a0_ref, a1_ref, g_ref, b_ref, w_ref, o_ref):
    dv = _dinv_col(ht_ref)
    t = dv * (a0_ref[0] + a1_ref[0] + g_ref[...]) + b_ref[...]
    t = jnp.maximum(t, 0.0)
    o_ref[...] = jnp.dot(t, w_ref[...],
                         preferred_element_type=jnp.float32) * dv


_layer2 = pl.pallas_call(
    _layer2_body,
    grid=(GRIDR,),
    in_specs=[_ht_spec(), _acc_spec(0), _acc_spec(1), _node_spec(),
              pl.BlockSpec((1, D), lambda i: (0, 0)),
              pl.BlockSpec((D, D), lambda i: (0, 0))],
    out_specs=_node_spec(),
    out_shape=jax.ShapeDtypeStruct((N, D), jnp.float32),
)


def _fin_body(ht_ref, a0_ref, a1_ref, g_ref, b_ref, o_ref):
    dv = _dinv_col(ht_ref)
    o_ref[...] = dv * (a0_ref[0] + a1_ref[0] + g_ref[...]) + b_ref[...]


_fin = pl.pallas_call(
    _fin_body,
    grid=(GRIDR,),
    in_specs=[_ht_spec(), _acc_spec(0), _acc_spec(1), _node_spec(),
              pl.BlockSpec((1, D), lambda i: (0, 0))],
    out_specs=_node_spec(),
    out_shape=jax.ShapeDtypeStruct((N, D), jnp.float32),
)


def kernel(x, edge_index, W1, b1, W2, b2):
    src = edge_index[0].astype(jnp.int32)
    dst = edge_index[1].astype(jnp.int32)
    pad = EPAD - src.shape[0]
    src_p = jnp.concatenate([src, jnp.arange(pad, dtype=jnp.int32) % N])
    fill = N + (jnp.arange(pad, dtype=jnp.int32) % (NPAD - N))
    dst_p = jnp.concatenate([dst, fill])

    def _sched(e):
        return e.reshape(NCH, CH)[_PERM].reshape(NW, K, CH)

    e4 = jnp.stack([_sched(src_p), _sched(dst_p)],
                   axis=2).reshape(NW, NB, BCH, 2, CH)

    zrows = jnp.zeros((ROWS_PER_TILE, D), jnp.float32)
    b1r = b1.reshape(1, D)
    b2r = b2.reshape(1, D)

    hist = _deg_kernel(dst_p)
    ht = hist.T
    g1 = _mm1(ht, x, W1)
    acc1 = _agg_kernel(g1, e4, zrows)
    g2 = _layer2(ht, acc1, acc1, g1, b1r, W2)
    acc2 = _agg_kernel(g2, e4, zrows)
    return _fin(ht, acc2, acc2, g2, b2r)
